# baseline (speedup 1.0000x reference)
import functools

import jax
import jax.numpy as jnp
from jax import lax
from jax.experimental import pallas as pl
from jax.experimental.pallas import tpu as pltpu
from jax.experimental.pallas import tpu_sc as plsc

DIM = 32
B = 16384
EPS = 1e-5

NC = 2
NS = 16
NW = NC * NS
BPW = B // NW
CH = 128
NCH = BPW // CH

_mesh = plsc.VectorSubcoreMesh(core_axis_name="c", subcore_axis_name="s")


@functools.partial(
    pl.kernel,
    mesh=_mesh,
    compiler_params=pltpu.CompilerParams(use_tc_tiling_on_sc=False),
    out_type=jax.ShapeDtypeStruct((B, DIM), jnp.float32),
    scratch_types=[
        pltpu.VMEM((NCH, CH), jnp.int32),
        pltpu.VMEM((BPW, DIM), jnp.float32),
        pltpu.SemaphoreType.DMA,
    ],
)
def _sc_gather(idx_hbm, table_hbm, out_hbm, idx_v, rows_v, sem):
    wid = lax.axis_index("s") * NC + lax.axis_index("c")
    pltpu.sync_copy(idx_hbm.at[wid], idx_v)
    copies = [
        pltpu.async_copy(
            table_hbm.at[idx_v.at[k]], rows_v.at[pl.ds(k * CH, CH)], sem
        )
        for k in range(NCH)
    ]
    for c in copies:
        c.wait()
    pltpu.sync_copy(rows_v, out_hbm.at[pl.ds(wid * BPW, BPW)])


CHUNK = 2048


def _tc_bn_elu(x_ref, g_ref, b_ref, o_ref):
    def stat_body(i, carry):
        s, q = carry
        xb = x_ref[:, pl.ds(i * CHUNK, CHUNK)]
        s = s + jnp.sum(xb, axis=1, keepdims=True)
        q = q + jnp.sum(xb * xb, axis=1, keepdims=True)
        return s, q

    zero = jnp.zeros((DIM, 1), jnp.float32)
    s, q = lax.fori_loop(0, B // CHUNK, stat_body, (zero, zero))
    mean = s * (1.0 / B)
    var = q * (1.0 / B) - mean * mean
    inv = lax.rsqrt(var + EPS)
    scale = g_ref[...] * inv
    shift = b_ref[...] - mean * scale

    def out_body(i, carry):
        xb = x_ref[:, pl.ds(i * CHUNK, CHUNK)]
        y = xb * scale + shift
        o_ref[:, pl.ds(i * CHUNK, CHUNK)] = jnp.where(y > 0, y, jnp.exp(y) - 1.0)
        return carry

    lax.fori_loop(0, B // CHUNK, out_body, 0)


def kernel(x, table, gamma, beta):
    idx = x.astype(jnp.int32).reshape(NW, NCH, CH)
    emb = _sc_gather(idx, table)
    out_t = pl.pallas_call(
        _tc_bn_elu,
        out_shape=jax.ShapeDtypeStruct((DIM, B), jnp.float32),
    )(emb.T, gamma.reshape(DIM, 1), beta.reshape(DIM, 1))
    return out_t.T


# speedup vs baseline: 1.4978x; 1.4978x over previous
import functools

import jax
import jax.numpy as jnp
from jax import lax
from jax.experimental import pallas as pl
from jax.experimental.pallas import tpu as pltpu
from jax.experimental.pallas import tpu_sc as plsc

DIM = 32
B = 16384
EPS = 1e-5
VOCAB = 1000000
CPAD = 1000064
W1 = 4096

NC = 2
NS = 16
NW = NC * NS
BPW = B // NW
CH = 128
NCH = BPW // CH

_mesh = plsc.VectorSubcoreMesh(core_axis_name="c", subcore_axis_name="s")


@functools.partial(
    pl.kernel,
    mesh=_mesh,
    out_type=jax.ShapeDtypeStruct((DIM * CPAD,), jnp.float32),
    scratch_types=[
        pltpu.VMEM((1, W1), jnp.float32),
    ],
)
def _sc_flatten(table_hbm, flat_hbm, buf):
    wid = lax.axis_index("s") * NC + lax.axis_index("c")

    def chunk(ci, carry):
        c0 = jnp.minimum(ci * W1, CPAD - W1)
        pltpu.sync_copy(table_hbm.at[pl.ds(wid, 1), pl.ds(c0, W1)], buf)
        pltpu.sync_copy(
            buf.at[0, pl.ds(0, W1)],
            flat_hbm.at[pl.ds(wid * CPAD + c0, W1)],
        )
        return carry

    lax.fori_loop(0, (CPAD + W1 - 1) // W1, chunk, 0)


@functools.partial(
    pl.kernel,
    mesh=_mesh,
    compiler_params=pltpu.CompilerParams(use_tc_tiling_on_sc=False),
    out_type=jax.ShapeDtypeStruct((DIM, B), jnp.float32),
    scratch_types=[
        pltpu.VMEM((BPW,), jnp.int32),
        pltpu.VMEM((BPW,), jnp.int32),
        pltpu.VMEM((1, BPW), jnp.float32),
        pltpu.SemaphoreType.DMA,
    ],
)
def _sc_gather(idx_hbm, flat_hbm, out_hbm, idx_v, off_v, row_v, sem):
    wid = lax.axis_index("s") * NC + lax.axis_index("c")
    base = wid * BPW
    pltpu.sync_copy(idx_hbm.at[pl.ds(base, BPW)], idx_v)
    nvec = BPW // 16

    def per_dim(d, carry):
        dbase = d * CPAD
        for k in range(nvec):
            off_v[pl.ds(k * 16, 16)] = idx_v[pl.ds(k * 16, 16)] + dbase
        copies = [
            pltpu.async_copy(
                flat_hbm.at[off_v.at[pl.ds(k * CH, CH)]],
                row_v.at[0, pl.ds(k * CH, CH)],
                sem,
            )
            for k in range(NCH)
        ]
        for c in copies:
            c.wait()
        pltpu.sync_copy(row_v, out_hbm.at[pl.ds(d, 1), pl.ds(base, BPW)])
        return carry

    lax.fori_loop(0, DIM, per_dim, 0)


CHUNK = 2048


def _tc_bn_elu(x_ref, g_ref, b_ref, o_ref):
    def stat_body(i, carry):
        s, q = carry
        xb = x_ref[:, pl.ds(i * CHUNK, CHUNK)]
        s = s + jnp.sum(xb, axis=1, keepdims=True)
        q = q + jnp.sum(xb * xb, axis=1, keepdims=True)
        return s, q

    zero = jnp.zeros((DIM, 1), jnp.float32)
    s, q = lax.fori_loop(0, B // CHUNK, stat_body, (zero, zero))
    mean = s * (1.0 / B)
    var = q * (1.0 / B) - mean * mean
    inv = lax.rsqrt(var + EPS)
    scale = g_ref[...] * inv
    shift = b_ref[...] - mean * scale

    def out_body(i, carry):
        xb = x_ref[:, pl.ds(i * CHUNK, CHUNK)]
        y = xb * scale + shift
        o_ref[:, pl.ds(i * CHUNK, CHUNK)] = jnp.where(y > 0, y, jnp.exp(y) - 1.0)
        return carry

    lax.fori_loop(0, B // CHUNK, out_body, 0)


def kernel(x, table, gamma, beta):
    idx = x.astype(jnp.int32)
    flat = _sc_flatten(table.T)
    emb_t = _sc_gather(idx, flat)
    out_t = pl.pallas_call(
        _tc_bn_elu,
        out_shape=jax.ShapeDtypeStruct((DIM, B), jnp.float32),
    )(emb_t, gamma.reshape(DIM, 1), beta.reshape(DIM, 1))
    return out_t.T


# speedup vs baseline: 2.0514x; 1.3696x over previous
import functools

import jax
import jax.numpy as jnp
from jax import lax
from jax.experimental import pallas as pl
from jax.experimental.pallas import tpu as pltpu
from jax.experimental.pallas import tpu_sc as plsc

DIM = 32
B = 16384
EPS = 1e-5
VOCAB = 1000000
CPAD = 1000064
W1 = 6144
NIT = 164

NC = 2
NS = 16
NW = NC * NS
BPW = B // NW
CH = 128
NCH = BPW // CH

_mesh = plsc.VectorSubcoreMesh(core_axis_name="c", subcore_axis_name="s")


@functools.partial(
    pl.kernel,
    mesh=_mesh,
    out_type=jax.ShapeDtypeStruct((DIM * CPAD,), jnp.float32),
    scratch_types=[
        pltpu.VMEM((1, W1), jnp.float32),
        pltpu.VMEM((1, W1), jnp.float32),
        pltpu.SemaphoreType.DMA,
        pltpu.SemaphoreType.DMA,
        pltpu.SemaphoreType.DMA,
        pltpu.SemaphoreType.DMA,
    ],
)
def _sc_flatten(table_hbm, flat_hbm, buf_a, buf_b, sia, sib, soa, sob):
    wid = lax.axis_index("s") * NC + lax.axis_index("c")

    def src(c):
        return table_hbm.at[pl.ds(wid, 1), pl.ds(c, W1)]

    def dst(c):
        return flat_hbm.at[pl.ds(wid * CPAD + c, W1)]

    def c_of(i):
        return jnp.minimum(i * W1, CPAD - W1)

    pltpu.async_copy(src(c_of(0)), buf_a, sia)

    def body(g, carry):
        i0 = 2 * g
        c0, c1, c2 = c_of(i0), c_of(i0 + 1), c_of(i0 + 2)
        pltpu.make_async_copy(src(c0), buf_a, sia).wait()
        pltpu.async_copy(buf_a.at[0, pl.ds(0, W1)], dst(c0), soa)
        pltpu.async_copy(src(c1), buf_b, sib)
        pltpu.make_async_copy(src(c1), buf_b, sib).wait()
        pltpu.async_copy(buf_b.at[0, pl.ds(0, W1)], dst(c1), sob)
        pltpu.make_async_copy(buf_a.at[0, pl.ds(0, W1)], dst(c0), soa).wait()
        pltpu.async_copy(src(c2), buf_a, sia)
        pltpu.make_async_copy(buf_b.at[0, pl.ds(0, W1)], dst(c1), sob).wait()
        return carry

    lax.fori_loop(0, NIT // 2, body, 0)
    pltpu.make_async_copy(src(c_of(NIT)), buf_a, sia).wait()


@functools.partial(
    pl.kernel,
    mesh=_mesh,
    compiler_params=pltpu.CompilerParams(use_tc_tiling_on_sc=False),
    out_type=jax.ShapeDtypeStruct((DIM, B), jnp.float32),
    scratch_types=[
        pltpu.VMEM((BPW,), jnp.int32),
        pltpu.VMEM((BPW,), jnp.int32),
        pltpu.VMEM((1, BPW), jnp.float32),
        pltpu.SemaphoreType.DMA,
    ],
)
def _sc_gather(idx_hbm, flat_hbm, out_hbm, idx_v, off_v, row_v, sem):
    wid = lax.axis_index("s") * NC + lax.axis_index("c")
    base = wid * BPW
    pltpu.sync_copy(idx_hbm.at[pl.ds(base, BPW)], idx_v)
    nvec = BPW // 16

    def per_dim(d, carry):
        dbase = d * CPAD
        for k in range(nvec):
            off_v[pl.ds(k * 16, 16)] = idx_v[pl.ds(k * 16, 16)] + dbase
        copies = [
            pltpu.async_copy(
                flat_hbm.at[off_v.at[pl.ds(k * CH, CH)]],
                row_v.at[0, pl.ds(k * CH, CH)],
                sem,
            )
            for k in range(NCH)
        ]
        for c in copies:
            c.wait()
        pltpu.sync_copy(row_v, out_hbm.at[pl.ds(d, 1), pl.ds(base, BPW)])
        return carry

    lax.fori_loop(0, DIM, per_dim, 0)


CHUNK = 2048


def _tc_bn_elu(x_ref, g_ref, b_ref, o_ref):
    def stat_body(i, carry):
        s, q = carry
        xb = x_ref[:, pl.ds(i * CHUNK, CHUNK)]
        s = s + jnp.sum(xb, axis=1, keepdims=True)
        q = q + jnp.sum(xb * xb, axis=1, keepdims=True)
        return s, q

    zero = jnp.zeros((DIM, 1), jnp.float32)
    s, q = lax.fori_loop(0, B // CHUNK, stat_body, (zero, zero))
    mean = s * (1.0 / B)
    var = q * (1.0 / B) - mean * mean
    inv = lax.rsqrt(var + EPS)
    scale = g_ref[...] * inv
    shift = b_ref[...] - mean * scale

    def out_body(i, carry):
        xb = x_ref[:, pl.ds(i * CHUNK, CHUNK)]
        y = xb * scale + shift
        o_ref[:, pl.ds(i * CHUNK, CHUNK)] = jnp.where(y > 0, y, jnp.exp(y) - 1.0)
        return carry

    lax.fori_loop(0, B // CHUNK, out_body, 0)


def kernel(x, table, gamma, beta):
    idx = x.astype(jnp.int32)
    flat = _sc_flatten(table.T)
    emb_t = _sc_gather(idx, flat)
    out_t = pl.pallas_call(
        _tc_bn_elu,
        out_shape=jax.ShapeDtypeStruct((DIM, B), jnp.float32),
    )(emb_t, gamma.reshape(DIM, 1), beta.reshape(DIM, 1))
    return out_t.T


# speedup vs baseline: 2.1511x; 1.0486x over previous
import functools

import jax
import jax.numpy as jnp
from jax import lax
from jax.experimental import pallas as pl
from jax.experimental.pallas import tpu as pltpu
from jax.experimental.pallas import tpu_sc as plsc

DIM = 32
B = 16384
EPS = 1e-5
VOCAB = 1000000
CPAD = 1000064
W1 = 6144
NIT = 164

NC = 2
NS = 16
NW = NC * NS
BPW = B // NW
CH = 128
NCH = BPW // CH

_mesh = plsc.VectorSubcoreMesh(core_axis_name="c", subcore_axis_name="s")


@functools.partial(
    pl.kernel,
    mesh=_mesh,
    out_type=jax.ShapeDtypeStruct((DIM * CPAD,), jnp.float32),
    scratch_types=[
        pltpu.VMEM((1, W1), jnp.float32),
        pltpu.VMEM((1, W1), jnp.float32),
        pltpu.SemaphoreType.DMA,
        pltpu.SemaphoreType.DMA,
        pltpu.SemaphoreType.DMA,
        pltpu.SemaphoreType.DMA,
    ],
)
def _sc_flatten(table_hbm, flat_hbm, buf_a, buf_b, sia, sib, soa, sob):
    wid = lax.axis_index("s") * NC + lax.axis_index("c")

    def src(c):
        return table_hbm.at[pl.ds(wid, 1), pl.ds(c, W1)]

    def dst(c):
        return flat_hbm.at[pl.ds(wid * CPAD + c, W1)]

    def c_of(i):
        return jnp.minimum(i * W1, CPAD - W1)

    pltpu.async_copy(src(c_of(0)), buf_a, sia)

    def body(g, carry):
        i0 = 2 * g
        c0, c1, c2 = c_of(i0), c_of(i0 + 1), c_of(i0 + 2)
        pltpu.make_async_copy(src(c0), buf_a, sia).wait()
        pltpu.async_copy(buf_a.at[0, pl.ds(0, W1)], dst(c0), soa)
        pltpu.async_copy(src(c1), buf_b, sib)
        pltpu.make_async_copy(src(c1), buf_b, sib).wait()
        pltpu.async_copy(buf_b.at[0, pl.ds(0, W1)], dst(c1), sob)
        pltpu.make_async_copy(buf_a.at[0, pl.ds(0, W1)], dst(c0), soa).wait()
        pltpu.async_copy(src(c2), buf_a, sia)
        pltpu.make_async_copy(buf_b.at[0, pl.ds(0, W1)], dst(c1), sob).wait()
        return carry

    lax.fori_loop(0, NIT // 2, body, 0)
    pltpu.make_async_copy(src(c_of(NIT)), buf_a, sia).wait()


@functools.partial(
    pl.kernel,
    mesh=_mesh,
    compiler_params=pltpu.CompilerParams(use_tc_tiling_on_sc=False),
    out_type=jax.ShapeDtypeStruct((DIM, B), jnp.float32),
    scratch_types=[
        pltpu.VMEM((BPW,), jnp.int32),
        pltpu.VMEM((DIM * BPW,), jnp.int32),
        pltpu.VMEM((1, BPW), jnp.float32),
        pltpu.VMEM((1, BPW), jnp.float32),
        pltpu.SemaphoreType.DMA,
        pltpu.SemaphoreType.DMA,
        pltpu.SemaphoreType.DMA,
        pltpu.SemaphoreType.DMA,
    ],
)
def _sc_gather(idx_hbm, flat_hbm, out_hbm, idx_v, off_v, row_a, row_b, sga, sgb, soa, sob):
    wid = lax.axis_index("s") * NC + lax.axis_index("c")
    base = wid * BPW
    pltpu.sync_copy(idx_hbm.at[pl.ds(base, BPW)], idx_v)
    nvec = BPW // 16
    for d in range(DIM):
        for k in range(nvec):
            off_v[pl.ds(d * BPW + k * 16, 16)] = idx_v[pl.ds(k * 16, 16)] + d * CPAD

    rows = (row_a, row_b)
    gsems = (sga, sgb)
    osems = (soa, sob)

    def gathers(d, buf, sem):
        return [
            pltpu.async_copy(
                flat_hbm.at[off_v.at[pl.ds(d * BPW + k * CH, CH)]],
                buf.at[0, pl.ds(k * CH, CH)],
                sem,
            )
            for k in range(NCH)
        ]

    def out_copy(d, buf, sem):
        return pltpu.async_copy(buf, out_hbm.at[pl.ds(d, 1), pl.ds(base, BPW)], sem)

    gathers(0, rows[0], gsems[0])
    for d in range(DIM):
        b = d % 2
        if d >= 1:
            pltpu.make_async_copy(
                rows[1 - b], out_hbm.at[pl.ds(d - 1, 1), pl.ds(base, BPW)], osems[1 - b]
            ).wait()
        if d + 1 < DIM:
            gathers(d + 1, rows[1 - b], gsems[1 - b])
        for k in range(NCH):
            pltpu.make_async_copy(
                flat_hbm.at[off_v.at[pl.ds(d * BPW + k * CH, CH)]],
                rows[b].at[0, pl.ds(k * CH, CH)],
                gsems[b],
            ).wait()
        out_copy(d, rows[b], osems[b])
    pltpu.make_async_copy(
        rows[1], out_hbm.at[pl.ds(DIM - 1, 1), pl.ds(base, BPW)], osems[1]
    ).wait()


CHUNK = 2048


def _tc_bn_elu(x_ref, g_ref, b_ref, o_ref):
    def stat_body(i, carry):
        s, q = carry
        xb = x_ref[:, pl.ds(i * CHUNK, CHUNK)]
        s = s + jnp.sum(xb, axis=1, keepdims=True)
        q = q + jnp.sum(xb * xb, axis=1, keepdims=True)
        return s, q

    zero = jnp.zeros((DIM, 1), jnp.float32)
    s, q = lax.fori_loop(0, B // CHUNK, stat_body, (zero, zero))
    mean = s * (1.0 / B)
    var = q * (1.0 / B) - mean * mean
    inv = lax.rsqrt(var + EPS)
    scale = g_ref[...] * inv
    shift = b_ref[...] - mean * scale

    def out_body(i, carry):
        xb = x_ref[:, pl.ds(i * CHUNK, CHUNK)]
        y = xb * scale + shift
        o_ref[:, pl.ds(i * CHUNK, CHUNK)] = jnp.where(y > 0, y, jnp.exp(y) - 1.0)
        return carry

    lax.fori_loop(0, B // CHUNK, out_body, 0)


def kernel(x, table, gamma, beta):
    idx = x.astype(jnp.int32)
    flat = _sc_flatten(table.T)
    emb_t = _sc_gather(idx, flat)
    out_t = pl.pallas_call(
        _tc_bn_elu,
        out_shape=jax.ShapeDtypeStruct((DIM, B), jnp.float32),
    )(emb_t, gamma.reshape(DIM, 1), beta.reshape(DIM, 1))
    return out_t.T
